# baseline (speedup 1.0000x reference)
import jax
import jax.numpy as jnp
from jax.experimental import pallas as pl
from jax.experimental.pallas import tpu as pltpu

_SLOTS = 4
_CHUNK = 15616
_N_FULL = 64
_TAIL = 1_000_000 - _N_FULL * _CHUNK


def _matcol(w_ref, b_ref, x):
    prod = jax.lax.dot_general(
        w_ref[...], x, (((0,), (0,)), ((), ())),
        preferred_element_type=jnp.float32,
    )
    return prod + jnp.transpose(b_ref[...], (1, 0))


def _affine_kernel(w_ref, b_ref, tT_hbm, outT_hbm, in_buf, out_buf,
                   tail_in, tail_out, in_sems, out_sems, tail_sems):
    i = pl.program_id(0)
    slot = jax.lax.rem(i, _SLOTS)

    class _Pair:
        def __init__(self, a, b):
            self._a, self._b = a, b

        def start(self):
            self._a.start()
            self._b.start()

        def wait(self):
            self._a.wait()
            self._b.wait()

    def in_copy(chunk_idx, buf_slot):
        return _Pair(*[
            pltpu.make_async_copy(
                tT_hbm.at[pl.ds(h * 32, 32),
                          pl.ds(chunk_idx * _CHUNK, _CHUNK)],
                in_buf.at[buf_slot, pl.ds(h * 32, 32), :],
                in_sems.at[buf_slot, h],
            )
            for h in (0, 1)
        ])

    def out_copy(buf_slot, chunk_idx):
        return _Pair(*[
            pltpu.make_async_copy(
                out_buf.at[buf_slot, pl.ds(h * 32, 32), :],
                outT_hbm.at[pl.ds(h * 32, 32),
                            pl.ds(chunk_idx * _CHUNK, _CHUNK)],
                out_sems.at[buf_slot, h],
            )
            for h in (0, 1)
        ])

    def tail_in_copy():
        return pltpu.make_async_copy(
            tT_hbm.at[:, pl.ds(_N_FULL * _CHUNK, _TAIL)],
            tail_in,
            tail_sems.at[0],
        )

    def tail_out_copy():
        return pltpu.make_async_copy(
            tail_out,
            outT_hbm.at[:, pl.ds(_N_FULL * _CHUNK, _TAIL)],
            tail_sems.at[1],
        )

    @pl.when(i == 0)
    def _():
        tail_in_copy().start()
        for k in range(min(_SLOTS - 1, _N_FULL)):
            in_copy(k, k % _SLOTS).start()
        tail_in_copy().wait()
        tail_out[...] = _matcol(w_ref, b_ref, tail_in[...])
        tail_out_copy().start()

    pre = i + _SLOTS - 1

    @pl.when(pre < _N_FULL)
    def _():
        in_copy(pre, jax.lax.rem(pre, _SLOTS)).start()

    in_copy(i, slot).wait()

    @pl.when(i >= _SLOTS)
    def _():
        out_copy(slot, i - _SLOTS).wait()

    out_buf[slot] = _matcol(w_ref, b_ref, in_buf[slot])
    out_copy(slot, i).start()

    @pl.when(i == _N_FULL - 1)
    def _():
        for k in range(_N_FULL - _SLOTS, _N_FULL):
            out_copy(k % _SLOTS, k).wait()
        tail_out_copy().wait()


def kernel(dummy, table, W, b):
    M, D = table.shape
    tT = table.T
    b_row = b.reshape(1, D)

    outT = pl.pallas_call(
        _affine_kernel,
        grid=(_N_FULL,),
        in_specs=[
            pl.BlockSpec((D, D), lambda i: (0, 0)),
            pl.BlockSpec((1, D), lambda i: (0, 0)),
            pl.BlockSpec(memory_space=pltpu.MemorySpace.HBM),
        ],
        out_specs=pl.BlockSpec(memory_space=pltpu.MemorySpace.HBM),
        out_shape=jax.ShapeDtypeStruct((D, M), jnp.float32),
        scratch_shapes=[
            pltpu.VMEM((_SLOTS, D, _CHUNK), jnp.float32),
            pltpu.VMEM((_SLOTS, D, _CHUNK), jnp.float32),
            pltpu.VMEM((D, _TAIL), jnp.float32),
            pltpu.VMEM((D, _TAIL), jnp.float32),
            pltpu.SemaphoreType.DMA((_SLOTS, 2)),
            pltpu.SemaphoreType.DMA((_SLOTS, 2)),
            pltpu.SemaphoreType.DMA((2,)),
        ],
        compiler_params=pltpu.CompilerParams(
            dimension_semantics=("arbitrary",),
        ),
    )(W, b_row, tT)
    return outT.T


# speedup vs baseline: 1.0055x; 1.0055x over previous
import jax
import jax.numpy as jnp
from jax.experimental import pallas as pl
from jax.experimental.pallas import tpu as pltpu

_SLOTS = 5
_CHUNK = 15616
_N_FULL = 64
_TAIL = 1_000_000 - _N_FULL * _CHUNK


def _matcol(w_ref, b_ref, x):
    prod = jax.lax.dot_general(
        w_ref[...], x, (((0,), (0,)), ((), ())),
        preferred_element_type=jnp.float32,
    )
    return prod + jnp.transpose(b_ref[...], (1, 0))


def _affine_kernel(w_ref, b_ref, tT_hbm, outT_hbm, in_buf, out_buf,
                   tail_in, tail_out, in_sems, out_sems, tail_sems):
    i = pl.program_id(0)
    slot = jax.lax.rem(i, _SLOTS)

    def in_copy(chunk_idx, buf_slot):
        return pltpu.make_async_copy(
            tT_hbm.at[:, pl.ds(chunk_idx * _CHUNK, _CHUNK)],
            in_buf.at[buf_slot],
            in_sems.at[buf_slot],
        )

    def out_copy(buf_slot, chunk_idx):
        return pltpu.make_async_copy(
            out_buf.at[buf_slot],
            outT_hbm.at[:, pl.ds(chunk_idx * _CHUNK, _CHUNK)],
            out_sems.at[buf_slot],
        )

    def tail_in_copy():
        return pltpu.make_async_copy(
            tT_hbm.at[:, pl.ds(_N_FULL * _CHUNK, _TAIL)],
            tail_in,
            tail_sems.at[0],
        )

    def tail_out_copy():
        return pltpu.make_async_copy(
            tail_out,
            outT_hbm.at[:, pl.ds(_N_FULL * _CHUNK, _TAIL)],
            tail_sems.at[1],
        )

    @pl.when(i == 0)
    def _():
        tail_in_copy().start()
        for k in range(min(_SLOTS - 1, _N_FULL)):
            in_copy(k, k % _SLOTS).start()
        tail_in_copy().wait()
        tail_out[...] = _matcol(w_ref, b_ref, tail_in[...])
        tail_out_copy().start()

    pre = i + _SLOTS - 1

    @pl.when(pre < _N_FULL)
    def _():
        in_copy(pre, jax.lax.rem(pre, _SLOTS)).start()

    in_copy(i, slot).wait()

    @pl.when(i >= _SLOTS)
    def _():
        out_copy(slot, i - _SLOTS).wait()

    out_buf[slot] = _matcol(w_ref, b_ref, in_buf[slot])
    out_copy(slot, i).start()

    @pl.when(i == _N_FULL - 1)
    def _():
        for k in range(_N_FULL - _SLOTS, _N_FULL):
            out_copy(k % _SLOTS, k).wait()
        tail_out_copy().wait()


def kernel(dummy, table, W, b):
    M, D = table.shape
    tT = table.T
    b_row = b.reshape(1, D)

    outT = pl.pallas_call(
        _affine_kernel,
        grid=(_N_FULL,),
        in_specs=[
            pl.BlockSpec((D, D), lambda i: (0, 0)),
            pl.BlockSpec((1, D), lambda i: (0, 0)),
            pl.BlockSpec(memory_space=pltpu.MemorySpace.HBM),
        ],
        out_specs=pl.BlockSpec(memory_space=pltpu.MemorySpace.HBM),
        out_shape=jax.ShapeDtypeStruct((D, M), jnp.float32),
        scratch_shapes=[
            pltpu.VMEM((_SLOTS, D, _CHUNK), jnp.float32),
            pltpu.VMEM((_SLOTS, D, _CHUNK), jnp.float32),
            pltpu.VMEM((D, _TAIL), jnp.float32),
            pltpu.VMEM((D, _TAIL), jnp.float32),
            pltpu.SemaphoreType.DMA((_SLOTS,)),
            pltpu.SemaphoreType.DMA((_SLOTS,)),
            pltpu.SemaphoreType.DMA((2,)),
        ],
        compiler_params=pltpu.CompilerParams(
            dimension_semantics=("arbitrary",),
        ),
    )(W, b_row, tT)
    return outT.T
